# baseline (speedup 1.0000x reference)
import functools

import jax
import jax.numpy as jnp
from jax import lax
from jax.experimental import pallas as pl
from jax.experimental.pallas import tpu as pltpu
from jax.experimental.pallas import tpu_sc as plsc

_NC = 2
_NS = 16
_NW = _NC * _NS
_IC = 128


def _make_gather(V, D, B, S):
  ns = B // S
  n_streams = ns // _IC
  cpw = D // _NW
  mesh = plsc.VectorSubcoreMesh(core_axis_name="c", subcore_axis_name="s")

  @functools.partial(
      pl.kernel,
      mesh=mesh,
      compiler_params=pltpu.CompilerParams(use_tc_tiling_on_sc=False),
      out_type=jax.ShapeDtypeStruct((S, D, ns), jnp.float32),
      scratch_types=[
          pltpu.VMEM((ns,), jnp.int32),
          pltpu.VMEM((ns,), jnp.int32),
          pltpu.VMEM((cpw, ns), jnp.float32),
          pltpu.VMEM((cpw, ns), jnp.float32),
          pltpu.SemaphoreType.DMA,
          pltpu.SemaphoreType.DMA,
          pltpu.SemaphoreType.DMA,
          pltpu.SemaphoreType.DMA,
          pltpu.SemaphoreType.DMA,
      ],
  )
  def emb(tab_t, ids_t, out_hbm, idx_a, idx_b, val_a, val_b,
          isem, gsem_a, gsem_b, osem_a, osem_b):
    wid = lax.axis_index("s") * _NC + lax.axis_index("c")
    c0 = wid * cpw

    def fire_idx(s, idx_v):
      pltpu.async_copy(ids_t.at[s], idx_v, isem)

    def drain_idx(s, idx_v):
      pltpu.make_async_copy(ids_t.at[s], idx_v, isem).wait()

    def fire_g(idx_v, val, gsem):
      for k in range(cpw):
        for j in range(n_streams):
          pltpu.async_copy(
              tab_t.at[c0 + k].at[idx_v.at[pl.ds(j * _IC, _IC)]],
              val.at[k, pl.ds(j * _IC, _IC)], gsem)

    def drain_g(idx_v, val, gsem):
      for k in range(cpw):
        for j in range(n_streams):
          pltpu.make_async_copy(
              tab_t.at[c0 + k].at[idx_v.at[pl.ds(j * _IC, _IC)]],
              val.at[k, pl.ds(j * _IC, _IC)], gsem).wait()

    def fire_o(s, val, osem):
      pltpu.async_copy(val, out_hbm.at[s, pl.ds(c0, cpw), :], osem)

    def drain_o(s, val, osem):
      pltpu.make_async_copy(val, out_hbm.at[s, pl.ds(c0, cpw), :],
                            osem).wait()

    fire_idx(0, idx_a)
    drain_idx(0, idx_a)
    fire_g(idx_a, val_a, gsem_a)
    fire_idx(1, idx_b)

    n_pairs = S // 2

    def body(p, carry):
      s0 = 2 * p
      s1 = s0 + 1

      drain_idx(s1, idx_b)

      @pl.when(p > 0)
      def _():
        drain_o(s1 - 2, val_b, osem_b)

      fire_g(idx_b, val_b, gsem_b)

      drain_g(idx_a, val_a, gsem_a)
      fire_o(s0, val_a, osem_a)

      @pl.when(s0 + 2 < S)
      def _():
        fire_idx(s0 + 2, idx_a)
        drain_idx(s0 + 2, idx_a)

      @pl.when(p + 1 < n_pairs)
      def _():
        drain_o(s0, val_a, osem_a)
        fire_g(idx_a, val_a, gsem_a)

      drain_g(idx_b, val_b, gsem_b)
      fire_o(s1, val_b, osem_b)

      @pl.when(s1 + 2 < S)
      def _():
        fire_idx(s1 + 2, idx_b)

      return carry

    lax.fori_loop(0, n_pairs, body, 0)
    drain_o(S - 2, val_a, osem_a)
    drain_o(S - 1, val_b, osem_b)

  return emb


_CW = 896


def _make_passthrough(V, D):
  mesh = plsc.VectorSubcoreMesh(core_axis_name="c", subcore_axis_name="s")

  @functools.partial(
      pl.kernel,
      mesh=mesh,
      compiler_params=pltpu.CompilerParams(
          use_tc_tiling_on_sc=True, needs_layout_passes=False),
      out_type=jax.ShapeDtypeStruct((D, V), jnp.float32),
      scratch_types=[
          pltpu.VMEM((D, _CW), jnp.float32),
          pltpu.VMEM((D, _CW), jnp.float32),
          pltpu.SemaphoreType.DMA,
          pltpu.SemaphoreType.DMA,
          pltpu.SemaphoreType.DMA,
          pltpu.SemaphoreType.DMA,
      ],
  )
  def pcopy(tab_hbm, out_hbm, buf_a, buf_b, isem_a, isem_b, osem_a, osem_b):
    wid = lax.axis_index("s") * _NC + lax.axis_index("c")
    nfull = V // _CW

    def fire_in(c, buf, isem):
      pltpu.async_copy(tab_hbm.at[:, pl.ds(c * _CW, _CW)], buf, isem)

    def drain_in(c, buf, isem):
      pltpu.make_async_copy(tab_hbm.at[:, pl.ds(c * _CW, _CW)], buf,
                            isem).wait()

    def fire_out(c, buf, osem):
      pltpu.async_copy(buf, out_hbm.at[:, pl.ds(c * _CW, _CW)], osem)

    def drain_out(c, buf, osem):
      pltpu.make_async_copy(buf, out_hbm.at[:, pl.ds(c * _CW, _CW)],
                            osem).wait()

    n_my = (nfull - 1 - wid) // _NW + 1

    @pl.when(n_my > 0)
    def _():
      fire_in(wid, buf_a, isem_a)

      def body2(p, carry):
        i1 = 2 * p + 1
        c0 = wid + 2 * p * _NW
        c1 = c0 + _NW

        @pl.when(p > 0)
        def _():
          drain_out(c0, buf_b, osem_b)

        @pl.when(i1 < n_my)
        def _():
          fire_in(c1, buf_b, isem_b)

        drain_in(c0, buf_a, isem_a)
        fire_out(c0, buf_a, osem_a)

        @pl.when(i1 + 1 < n_my)
        def _():
          drain_out(c0, buf_a, osem_a)
          fire_in(c1 + _NW, buf_a, isem_a)

        @pl.when(i1 < n_my)
        def _():
          drain_in(c1, buf_b, isem_b)
          fire_out(c1, buf_b, osem_b)

        return carry

      np_ = (n_my + 1) // 2
      lax.fori_loop(0, np_, body2, 0)
      drain_out(wid, buf_a, osem_a)

      @pl.when(n_my % 2 == 0)
      def _():
        drain_out(wid, buf_b, osem_b)


  return pcopy


def kernel(input_ids, use_one_hot_embeddings, embedding_table):
  V, D = embedding_table.shape
  B, S = input_ids.shape
  ids_t = input_ids.T

  tab_copy = _make_passthrough(V, D)(embedding_table.T).T
  ids_t, tab_copy = lax.optimization_barrier((ids_t, tab_copy))
  out = _make_gather(V, D, B * S, S)(embedding_table.T, ids_t)
  out = jnp.transpose(out, (2, 0, 1))
  aligned = V // _CW * _CW
  if aligned < V:
    tab_copy = lax.dynamic_update_slice(
        tab_copy, embedding_table[aligned:, :], (aligned, 0))
  return (out, tab_copy)


# speedup vs baseline: 6.2669x; 6.2669x over previous
import functools

import jax
import jax.numpy as jnp
from jax import lax
from jax.experimental import pallas as pl
from jax.experimental.pallas import tpu as pltpu
from jax.experimental.pallas import tpu_sc as plsc

_NC = 2
_NS = 16
_NW = _NC * _NS
_CH = 128
_K = 5


def _make_gather(V, D, B):
  b_per_w = B // _NW
  n_chunks = b_per_w // _CH
  n_groups = n_chunks // _K
  n_pairs = n_groups // 2
  grp_rows = _K * _CH
  mesh = plsc.VectorSubcoreMesh(core_axis_name="c", subcore_axis_name="s")

  @functools.partial(
      pl.kernel,
      mesh=mesh,
      compiler_params=pltpu.CompilerParams(use_tc_tiling_on_sc=False),
      out_type=jax.ShapeDtypeStruct((B, D), jnp.float32),
      scratch_types=[
          pltpu.VMEM((n_chunks, _CH), jnp.int32),
          pltpu.VMEM((grp_rows, D), jnp.float32),
          pltpu.VMEM((grp_rows, D), jnp.float32),
          pltpu.SemaphoreType.DMA,
          pltpu.SemaphoreType.DMA,
          pltpu.SemaphoreType.DMA,
          pltpu.SemaphoreType.DMA,
      ],
  )
  def emb(table_hbm, idx_hbm, out_hbm, idx_v, rows_a, rows_b,
          gsem_a, gsem_b, osem_a, osem_b):
    wid = lax.axis_index("s") * _NC + lax.axis_index("c")
    base = wid * b_per_w
    pltpu.sync_copy(idx_hbm.at[wid], idx_v)

    def fire_gathers(g, rows, gsem):
      for b in range(_K):
        pltpu.async_copy(
            table_hbm.at[idx_v.at[g * _K + b]],
            rows.at[pl.ds(b * _CH, _CH)], gsem)

    def drain_gathers(g, rows, gsem):
      for b in range(_K):
        pltpu.make_async_copy(
            table_hbm.at[idx_v.at[g * _K + b]],
            rows.at[pl.ds(b * _CH, _CH)], gsem).wait()

    def fire_wb(g, rows, osem):
      pltpu.async_copy(rows, out_hbm.at[pl.ds(base + g * grp_rows, grp_rows)],
                       osem)

    def drain_wb(g, rows, osem):
      pltpu.make_async_copy(rows,
                            out_hbm.at[pl.ds(base + g * grp_rows, grp_rows)],
                            osem).wait()

    fire_gathers(0, rows_a, gsem_a)

    def body(p, carry):
      g0 = 2 * p
      g1 = g0 + 1

      @pl.when(p > 0)
      def _():
        drain_wb(g1, rows_b, osem_b)

      fire_gathers(g1, rows_b, gsem_b)
      drain_gathers(g0, rows_a, gsem_a)
      fire_wb(g0, rows_a, osem_a)

      @pl.when(p + 1 < n_pairs)
      def _():
        drain_wb(g0, rows_a, osem_a)
        fire_gathers(g0 + 2, rows_a, gsem_a)

      drain_gathers(g1, rows_b, gsem_b)
      fire_wb(g1, rows_b, osem_b)
      return carry

    lax.fori_loop(0, n_pairs, body, 0)
    drain_wb(0, rows_a, osem_a)
    drain_wb(0, rows_b, osem_b)

  return emb


_CW = 896


def _make_passthrough(V, D):
  ncols_pad = (V + 127) // 128 * 128
  total_chunks = ncols_pad // _CW if ncols_pad % _CW == 0 else ncols_pad // _CW + 1
  mesh = plsc.VectorSubcoreMesh(core_axis_name="c", subcore_axis_name="s")

  @functools.partial(
      pl.kernel,
      mesh=mesh,
      compiler_params=pltpu.CompilerParams(
          use_tc_tiling_on_sc=True, needs_layout_passes=False,
          disable_bounds_checks=True),
      out_type=jax.ShapeDtypeStruct((D, V), jnp.float32),
      scratch_types=[
          pltpu.VMEM((D, _CW), jnp.float32),
          pltpu.VMEM((D, _CW), jnp.float32),
          pltpu.SemaphoreType.DMA,
          pltpu.SemaphoreType.DMA,
          pltpu.SemaphoreType.DMA,
          pltpu.SemaphoreType.DMA,
      ],
  )
  def pcopy(tab_hbm, out_hbm, buf_a, buf_b, isem_a, isem_b, osem_a, osem_b):
    wid = lax.axis_index("s") * _NC + lax.axis_index("c")
    nfull = V // _CW
    def fire_in(c, buf, isem):
      pltpu.async_copy(tab_hbm.at[:, pl.ds(c * _CW, _CW)], buf, isem)

    def drain_in(c, buf, isem):
      pltpu.make_async_copy(tab_hbm.at[:, pl.ds(c * _CW, _CW)], buf,
                            isem).wait()

    def fire_out(c, buf, osem):
      pltpu.async_copy(buf, out_hbm.at[:, pl.ds(c * _CW, _CW)], osem)

    def drain_out(c, buf, osem):
      pltpu.make_async_copy(buf, out_hbm.at[:, pl.ds(c * _CW, _CW)],
                            osem).wait()

    n_my = (nfull - 1 - wid) // _NW + 1

    @pl.when(n_my > 0)
    def _():
      fire_in(wid, buf_a, isem_a)

      def body2(p, carry):
        i1 = 2 * p + 1
        c0 = wid + 2 * p * _NW
        c1 = c0 + _NW

        @pl.when(p > 0)
        def _():
          drain_out(c0, buf_b, osem_b)

        @pl.when(i1 < n_my)
        def _():
          fire_in(c1, buf_b, isem_b)

        drain_in(c0, buf_a, isem_a)
        fire_out(c0, buf_a, osem_a)

        @pl.when(i1 + 1 < n_my)
        def _():
          drain_out(c0, buf_a, osem_a)
          fire_in(c1 + _NW, buf_a, isem_a)

        @pl.when(i1 < n_my)
        def _():
          drain_in(c1, buf_b, isem_b)
          fire_out(c1, buf_b, osem_b)

        return carry

      np_ = (n_my + 1) // 2
      lax.fori_loop(0, np_, body2, 0)
      drain_out(wid, buf_a, osem_a)

      @pl.when(n_my % 2 == 0)
      def _():
        drain_out(wid, buf_b, osem_b)


  return pcopy


def kernel(input_ids, use_one_hot_embeddings, embedding_table):
  V, D = embedding_table.shape
  orig_shape = input_ids.shape
  flat = input_ids.reshape(-1)
  B = flat.shape[0]
  b_per_w = B // _NW
  n_chunks = b_per_w // _CH
  idx3 = flat.reshape(_NW, n_chunks, _CH)

  tab_copy = _make_passthrough(V, D)(embedding_table.T).T
  idx3, tab_copy = lax.optimization_barrier((idx3, tab_copy))
  out = _make_gather(V, D, B)(embedding_table, idx3)
  out = out.reshape(orig_shape + (D,))
  aligned = V // _CW * _CW
  if aligned < V:
    tab_copy = lax.dynamic_update_slice(
        tab_copy, embedding_table[aligned:, :], (aligned, 0))
  return (out, tab_copy)
